# baseline (speedup 1.0000x reference)
import functools

import jax
import jax.numpy as jnp
from jax import lax
from jax.experimental import pallas as pl
from jax.experimental.pallas import tpu as pltpu
from jax.experimental.pallas import tpu_sc as plsc

N = 10000
E = 320000
D_IN = 128
D_HID = 16
D_OUT = 40

NC = 2
NS = 16
NW = NC * NS

NPAD = 10240
RPT = NPAD // NS
CHUNK = 128
K = 8
CH = 80
EPW = CH * CHUNK
E_PAD = EPW * NW
CHA = E_PAD // (NS * CHUNK)

_mesh = plsc.VectorSubcoreMesh(
    core_axis_name="c", subcore_axis_name="s", num_cores=NC, num_subcores=NS)
_sc_params = pltpu.CompilerParams(use_tc_tiling_on_sc=False,
                                  needs_layout_passes=False)


def _rsqrt16(x):
    i = plsc.bitcast(x, jnp.int32)
    i = jnp.int32(0x5F3759DF) - lax.shift_right_logical(i, 1)
    y = plsc.bitcast(i, jnp.float32)
    for _ in range(3):
        y = y * (1.5 - 0.5 * x * y * y)
    return y


def _scatter_pass(src_v, dst_v, rows_v, ytab, atab, gsems):
    for b in range(K):
        pltpu.async_copy(ytab.at[src_v.at[b]], rows_v.at[b], gsems.at[b])

    def outer(jb, _):
        for b in range(K):
            j = jb * K + b
            pltpu.make_async_copy(
                ytab.at[src_v.at[j]], rows_v.at[b], gsems.at[b]).wait()
            pltpu.sync_copy(rows_v.at[b], atab.at[dst_v.at[j]], add=True)
            nj = j + K

            @pl.when(nj < CH)
            def _():
                pltpu.async_copy(
                    ytab.at[src_v.at[nj]], rows_v.at[b], gsems.at[b])
        return 0

    lax.fori_loop(0, CH // K, outer, 0)


@functools.partial(
    pl.kernel,
    out_type=(jax.ShapeDtypeStruct((NC, NPAD, D_HID), jnp.float32),
              jax.ShapeDtypeStruct((NPAD, D_HID), jnp.float32),
              jax.ShapeDtypeStruct((NPAD,), jnp.float32)),
    mesh=_mesh,
    scratch_types=[
        pltpu.VMEM((CH, CHUNK), jnp.int32),
        pltpu.VMEM((CH, CHUNK), jnp.int32),
        pltpu.VMEM((CHA, CHUNK), jnp.int32),
        pltpu.VMEM((CHUNK,), jnp.float32),
        pltpu.VMEM((RPT,), jnp.float32),
        pltpu.VMEM((RPT,), jnp.float32),
        pltpu.VMEM((RPT, D_HID), jnp.float32),
        pltpu.VMEM((K, CHUNK, D_HID), jnp.float32),
        pltpu.VMEM_SHARED((NPAD,), jnp.float32),
        pltpu.VMEM_SHARED((NPAD, D_HID), jnp.float32),
        pltpu.VMEM_SHARED((NPAD, D_HID), jnp.float32),
        pltpu.SemaphoreType.DMA((K,)),
        pltpu.SemaphoreType.DMA((K,)),
    ],
    compiler_params=_sc_params,
)
def _sc_layer1(src_hbm, dst_hbm, xw_hbm,
               parts_hbm, y1_hbm, dinv_hbm,
               src_v, dst_v, dsta_v, ones_v, deg_v, dinv_v, y_v, rows_v,
               degtab, ytab, atab, gsems, hsems):
    cid = lax.axis_index("c")
    sid = lax.axis_index("s")
    w = sid * NC + cid
    rs = sid * RPT

    def fill_ones(i, _):
        ones_v[pl.ds(i * 16, 16)] = jnp.ones((16,), jnp.float32)
        return 0

    lax.fori_loop(0, CHUNK // 16, fill_ones, 0)

    def fill_zeros(i, _):
        deg_v[pl.ds(i * 16, 16)] = jnp.zeros((16,), jnp.float32)
        return 0

    lax.fori_loop(0, RPT // 16, fill_zeros, 0)

    pltpu.sync_copy(deg_v, degtab.at[pl.ds(rs, RPT)])
    pltpu.sync_copy(dst_hbm.at[2 * sid], dsta_v.at[pl.ds(0, CH)])
    pltpu.sync_copy(dst_hbm.at[2 * sid + 1], dsta_v.at[pl.ds(CH, CH)])
    pltpu.sync_copy(src_hbm.at[w], src_v)
    pltpu.sync_copy(dst_hbm.at[w], dst_v)
    pltpu.sync_copy(xw_hbm.at[pl.ds(rs, RPT)], y_v)
    plsc.subcore_barrier()

    for b in range(K):
        pltpu.async_copy(ones_v, degtab.at[dsta_v.at[b]], hsems.at[b],
                         add=True)

    def hist(jb, _):
        for b in range(K):
            j = jb * K + b
            pltpu.make_async_copy(
                ones_v, degtab.at[dsta_v.at[j]], hsems.at[b]).wait()
            nj = j + K

            @pl.when(nj < CHA)
            def _():
                pltpu.async_copy(ones_v, degtab.at[dsta_v.at[nj]],
                                 hsems.at[b], add=True)
        return 0

    lax.fori_loop(0, CHA // K, hist, 0)
    plsc.subcore_barrier()

    pltpu.sync_copy(degtab.at[pl.ds(rs, RPT)], deg_v)

    def mk_dinv(i, _):
        d = deg_v[pl.ds(i * 16, 16)]
        dinv_v[pl.ds(i * 16, 16)] = _rsqrt16(d + 1.0)
        return 0

    lax.fori_loop(0, RPT // 16, mk_dinv, 0)

    def scale_row(r, _):
        dv = plsc.load_gather(dinv_v, [jnp.full((16,), r, jnp.int32)])
        y_v[r, :] = dv * y_v[r, :]
        return 0

    lax.fori_loop(0, RPT, scale_row, 0)

    pltpu.sync_copy(y_v, ytab.at[pl.ds(rs, RPT)])
    pltpu.sync_copy(y_v, atab.at[pl.ds(rs, RPT)])

    @pl.when(cid == 0)
    def _():
        pltpu.sync_copy(y_v, y1_hbm.at[pl.ds(rs, RPT)])
        pltpu.sync_copy(dinv_v, dinv_hbm.at[pl.ds(rs, RPT)])

    plsc.subcore_barrier()
    _scatter_pass(src_v, dst_v, rows_v, ytab, atab, gsems)
    plsc.subcore_barrier()
    pltpu.sync_copy(atab.at[pl.ds(rs, RPT)],
                    parts_hbm.at[cid, pl.ds(rs, RPT)])


@functools.partial(
    pl.kernel,
    out_type=jax.ShapeDtypeStruct((NC, NPAD, D_HID), jnp.float32),
    mesh=_mesh,
    scratch_types=[
        pltpu.VMEM((CH, CHUNK), jnp.int32),
        pltpu.VMEM((CH, CHUNK), jnp.int32),
        pltpu.VMEM((K, CHUNK, D_HID), jnp.float32),
        pltpu.VMEM_SHARED((NPAD, D_HID), jnp.float32),
        pltpu.VMEM_SHARED((NPAD, D_HID), jnp.float32),
        pltpu.SemaphoreType.DMA((K,)),
    ],
    compiler_params=_sc_params,
)
def _sc_layer2(src_hbm, dst_hbm, y_hbm, out_hbm,
               src_v, dst_v, rows_v, ytab, atab, gsems):
    cid = lax.axis_index("c")
    sid = lax.axis_index("s")
    w = sid * NC + cid
    rs = sid * RPT

    pltpu.sync_copy(y_hbm.at[pl.ds(rs, RPT)], ytab.at[pl.ds(rs, RPT)])
    pltpu.sync_copy(y_hbm.at[pl.ds(rs, RPT)], atab.at[pl.ds(rs, RPT)])
    pltpu.sync_copy(src_hbm.at[w], src_v)
    pltpu.sync_copy(dst_hbm.at[w], dst_v)
    plsc.subcore_barrier()
    _scatter_pass(src_v, dst_v, rows_v, ytab, atab, gsems)
    plsc.subcore_barrier()
    pltpu.sync_copy(atab.at[pl.ds(rs, RPT)],
                    out_hbm.at[cid, pl.ds(rs, RPT)])


def _mm_body(x_ref, w_ref, o_ref):
    o_ref[...] = jnp.dot(x_ref[...], w_ref[...],
                         preferred_element_type=jnp.float32)


def _mid_body(parts_ref, y_ref, dinv_ref, y2_ref):
    s = parts_ref[0] + parts_ref[1] - y_ref[...]
    d = dinv_ref[...]
    y2_ref[...] = (d * d) * s


def _final_body(parts_ref, y2_ref, dinv_ref, w2_ref, o_ref):
    s = parts_ref[0] + parts_ref[1] - y2_ref[...]
    g = dinv_ref[...] * s
    o_ref[...] = jnp.dot(g, w2_ref[...], preferred_element_type=jnp.float32)


def kernel(x, edge_index, W1, W2):
    f32 = jnp.float32
    x_pad = jnp.zeros((NPAD, D_IN), f32).at[:N].set(x)

    pad_cnt = E_PAD - E
    pad_idx = (N + jnp.arange(pad_cnt, dtype=jnp.int32) % (NPAD - N))
    src_flat = jnp.concatenate([edge_index[0], pad_idx])
    dst_flat = jnp.concatenate([edge_index[1], pad_idx])
    src = src_flat.reshape(NW, CH, CHUNK)
    dst = dst_flat.reshape(NW, CH, CHUNK)

    xw1 = pl.pallas_call(
        _mm_body,
        out_shape=jax.ShapeDtypeStruct((NPAD, D_HID), f32),
    )(x_pad, W1)

    parts1, y1, dinv = _sc_layer1(src, dst, xw1)
    dinv = dinv.reshape(NPAD, 1)

    y2 = pl.pallas_call(
        _mid_body,
        out_shape=jax.ShapeDtypeStruct((NPAD, D_HID), f32),
    )(parts1, y1, dinv)

    parts2 = _sc_layer2(src, dst, y2)

    out_pad = pl.pallas_call(
        _final_body,
        out_shape=jax.ShapeDtypeStruct((NPAD, D_OUT), f32),
    )(parts2, y2, dinv, W2)

    return out_pad[:N]


# speedup vs baseline: 1.2165x; 1.2165x over previous
import functools

import jax
import jax.numpy as jnp
from jax import lax
from jax.experimental import pallas as pl
from jax.experimental.pallas import tpu as pltpu
from jax.experimental.pallas import tpu_sc as plsc
from jax.scipy.linalg import block_diag

N = 10000
E = 320000
D_IN = 128
D_HID = 16
D_OUT = 40

NC = 2
NS = 16
NW = NC * NS

NPAD = 10240
RPT = NPAD // NS
FPT = RPT // 8
NF = NPAD // 8
XF = N // 8
CHUNK = 125
K = 8
CH = 80
CHA = 2 * CH

_mesh = plsc.VectorSubcoreMesh(
    core_axis_name="c", subcore_axis_name="s", num_cores=NC, num_subcores=NS)
_sc_params = pltpu.CompilerParams(use_tc_tiling_on_sc=False,
                                  needs_layout_passes=False)


def _rsqrt16(x):
    i = plsc.bitcast(x, jnp.int32)
    i = jnp.int32(0x5F3759DF) - lax.shift_right_logical(i, 1)
    y = plsc.bitcast(i, jnp.float32)
    for _ in range(3):
        y = y * (1.5 - 0.5 * x * y * y)
    return y


def _scatter_pass(src_v, dst_v, rows_v, ytab, atab, gsems):
    for b in range(K):
        pltpu.async_copy(ytab.at[src_v.at[b]], rows_v.at[b], gsems.at[b])

    def outer(jb, _):
        for b in range(K):
            j = jb * K + b
            pltpu.make_async_copy(
                ytab.at[src_v.at[j]], rows_v.at[b], gsems.at[b]).wait()
            pltpu.sync_copy(rows_v.at[b], atab.at[dst_v.at[j]], add=True)
            nj = j + K

            @pl.when(nj < CH)
            def _():
                pltpu.async_copy(
                    ytab.at[src_v.at[nj]], rows_v.at[b], gsems.at[b])
        return 0

    lax.fori_loop(0, CH // K, outer, 0)


def _repack_to_flat(n16_v, flat_v):
    def body(r, _):
        for g in range(8):
            flat_v[r, pl.ds(g * 16, 16)] = n16_v[r * 8 + g, :]
        return 0

    lax.fori_loop(0, FPT, body, 0)


@functools.partial(
    pl.kernel,
    out_type=(jax.ShapeDtypeStruct((NC, NF, 128), jnp.float32),
              jax.ShapeDtypeStruct((NF, 128), jnp.float32),
              jax.ShapeDtypeStruct((NF, 128), jnp.float32)),
    mesh=_mesh,
    scratch_types=[
        pltpu.VMEM((CH, CHUNK), jnp.int32),
        pltpu.VMEM((CH, CHUNK), jnp.int32),
        pltpu.VMEM((CHA, CHUNK), jnp.int32),
        pltpu.VMEM((128,), jnp.float32),
        pltpu.VMEM((RPT,), jnp.float32),
        pltpu.VMEM((RPT,), jnp.float32),
        pltpu.VMEM((FPT, 128), jnp.float32),
        pltpu.VMEM((FPT, 128), jnp.float32),
        pltpu.VMEM((RPT, D_HID), jnp.float32),
        pltpu.VMEM((K, CHUNK, D_HID), jnp.float32),
        pltpu.VMEM_SHARED((NPAD,), jnp.float32),
        pltpu.VMEM_SHARED((NPAD, D_HID), jnp.float32),
        pltpu.VMEM_SHARED((NPAD, D_HID), jnp.float32),
        pltpu.SemaphoreType.DMA((K,)),
        pltpu.SemaphoreType.DMA((K,)),
    ],
    compiler_params=_sc_params,
)
def _sc_layer1(src_hbm, dst_hbm, xw_hbm,
               parts_hbm, y1_hbm, dinv_hbm,
               src_v, dst_v, dsta_v, ones_v, deg_v, dinv_v, xwf_v, dbf_v,
               y16_v, rows_v, degtab, ytab, atab, gsems, hsems):
    cid = lax.axis_index("c")
    sid = lax.axis_index("s")
    w = sid * NC + cid
    rs = sid * RPT
    fs = sid * FPT

    def fill_ones(i, _):
        ones_v[pl.ds(i * 16, 16)] = jnp.ones((16,), jnp.float32)
        return 0

    lax.fori_loop(0, 8, fill_ones, 0)
    ones = ones_v.at[pl.ds(0, CHUNK)]

    def fill_zeros(i, _):
        deg_v[pl.ds(i * 16, 16)] = jnp.zeros((16,), jnp.float32)
        return 0

    lax.fori_loop(0, RPT // 16, fill_zeros, 0)

    pltpu.sync_copy(deg_v, degtab.at[pl.ds(rs, RPT)])
    pltpu.sync_copy(dst_hbm.at[2 * sid], dsta_v.at[pl.ds(0, CH)])
    pltpu.sync_copy(dst_hbm.at[2 * sid + 1], dsta_v.at[pl.ds(CH, CH)])
    pltpu.sync_copy(src_hbm.at[w], src_v)
    pltpu.sync_copy(dst_hbm.at[w], dst_v)
    pltpu.sync_copy(xw_hbm.at[pl.ds(fs, FPT)], xwf_v)
    plsc.subcore_barrier()

    for b in range(K):
        pltpu.async_copy(ones, degtab.at[dsta_v.at[b]], hsems.at[b],
                         add=True)

    def hist(jb, _):
        for b in range(K):
            j = jb * K + b
            pltpu.make_async_copy(
                ones, degtab.at[dsta_v.at[j]], hsems.at[b]).wait()
            nj = j + K

            @pl.when(nj < CHA)
            def _():
                pltpu.async_copy(ones, degtab.at[dsta_v.at[nj]],
                                 hsems.at[b], add=True)
        return 0

    lax.fori_loop(0, CHA // K, hist, 0)
    plsc.subcore_barrier()

    pltpu.sync_copy(degtab.at[pl.ds(rs, RPT)], deg_v)

    def mk_dinv(i, _):
        d = deg_v[pl.ds(i * 16, 16)]
        dinv_v[pl.ds(i * 16, 16)] = _rsqrt16(d + 1.0)
        return 0

    lax.fori_loop(0, RPT // 16, mk_dinv, 0)

    def scale_row(r, _):
        for g in range(8):
            n = r * 8 + g
            dv = plsc.load_gather(dinv_v, [jnp.full((16,), n, jnp.int32)])
            y = dv * xwf_v[r, pl.ds(g * 16, 16)]
            y16_v[n, :] = y
            xwf_v[r, pl.ds(g * 16, 16)] = y
            dbf_v[r, pl.ds(g * 16, 16)] = dv
        return 0

    lax.fori_loop(0, FPT, scale_row, 0)

    pltpu.sync_copy(y16_v, ytab.at[pl.ds(rs, RPT)])
    pltpu.sync_copy(y16_v, atab.at[pl.ds(rs, RPT)])

    @pl.when(cid == 0)
    def _():
        pltpu.sync_copy(xwf_v, y1_hbm.at[pl.ds(fs, FPT)])
        pltpu.sync_copy(dbf_v, dinv_hbm.at[pl.ds(fs, FPT)])

    plsc.subcore_barrier()
    _scatter_pass(src_v, dst_v, rows_v, ytab, atab, gsems)
    plsc.subcore_barrier()
    pltpu.sync_copy(atab.at[pl.ds(rs, RPT)], y16_v)
    _repack_to_flat(y16_v, xwf_v)
    pltpu.sync_copy(xwf_v, parts_hbm.at[cid, pl.ds(fs, FPT)])


@functools.partial(
    pl.kernel,
    out_type=(jax.ShapeDtypeStruct((NC, NF, 128), jnp.float32),
              jax.ShapeDtypeStruct((NF, 128), jnp.float32)),
    mesh=_mesh,
    scratch_types=[
        pltpu.VMEM((CH, CHUNK), jnp.int32),
        pltpu.VMEM((CH, CHUNK), jnp.int32),
        pltpu.VMEM((FPT, 128), jnp.float32),
        pltpu.VMEM((FPT, 128), jnp.float32),
        pltpu.VMEM((FPT, 128), jnp.float32),
        pltpu.VMEM((FPT, 128), jnp.float32),
        pltpu.VMEM((RPT, D_HID), jnp.float32),
        pltpu.VMEM((K, CHUNK, D_HID), jnp.float32),
        pltpu.VMEM_SHARED((NPAD, D_HID), jnp.float32),
        pltpu.VMEM_SHARED((NPAD, D_HID), jnp.float32),
        pltpu.SemaphoreType.DMA((K,)),
    ],
    compiler_params=_sc_params,
)
def _sc_layer2(src_hbm, dst_hbm, parts_hbm, y1_hbm, dinv_hbm,
               out_hbm, y2_hbm,
               src_v, dst_v, p0_v, p1_v, y1_v, db_v, y16_v, rows_v,
               ytab, atab, gsems):
    cid = lax.axis_index("c")
    sid = lax.axis_index("s")
    w = sid * NC + cid
    rs = sid * RPT
    fs = sid * FPT

    pltpu.sync_copy(src_hbm.at[w], src_v)
    pltpu.sync_copy(dst_hbm.at[w], dst_v)
    pltpu.sync_copy(parts_hbm.at[0, pl.ds(fs, FPT)], p0_v)
    pltpu.sync_copy(parts_hbm.at[1, pl.ds(fs, FPT)], p1_v)
    pltpu.sync_copy(y1_hbm.at[pl.ds(fs, FPT)], y1_v)
    pltpu.sync_copy(dinv_hbm.at[pl.ds(fs, FPT)], db_v)

    def mk_y2(r, _):
        for g in range(8):
            s = pl.ds(g * 16, 16)
            d = db_v[r, s]
            y2 = (d * d) * (p0_v[r, s] + p1_v[r, s] - y1_v[r, s])
            y16_v[r * 8 + g, :] = y2
            p0_v[r, s] = y2
        return 0

    lax.fori_loop(0, FPT, mk_y2, 0)

    pltpu.sync_copy(y16_v, ytab.at[pl.ds(rs, RPT)])
    pltpu.sync_copy(y16_v, atab.at[pl.ds(rs, RPT)])

    @pl.when(cid == 0)
    def _():
        pltpu.sync_copy(p0_v, y2_hbm.at[pl.ds(fs, FPT)])

    plsc.subcore_barrier()
    _scatter_pass(src_v, dst_v, rows_v, ytab, atab, gsems)
    plsc.subcore_barrier()
    pltpu.sync_copy(atab.at[pl.ds(rs, RPT)], y16_v)
    _repack_to_flat(y16_v, p0_v)
    pltpu.sync_copy(p0_v, out_hbm.at[cid, pl.ds(fs, FPT)])


def _mm1_body(x_ref, w_ref, o_ref):
    r = jnp.dot(x_ref[...], w_ref[...], preferred_element_type=jnp.float32)
    o_ref[...] = jnp.concatenate(
        [r, jnp.zeros((NF - XF, 128), jnp.float32)], axis=0)


def _final_body(parts_ref, y2_ref, dinv_ref, w2_ref, o_ref):
    g = dinv_ref[...] * (parts_ref[0] + parts_ref[1] - y2_ref[...])
    o_ref[...] = jnp.dot(g, w2_ref[...], preferred_element_type=jnp.float32)


def kernel(x, edge_index, W1, W2):
    f32 = jnp.float32
    x_flat = x.reshape(XF, 8 * D_IN)
    w1_big = block_diag(*([W1] * 8))
    w2_big = block_diag(*([W2] * 8))

    src = edge_index[0].reshape(NW, CH, CHUNK)
    dst = edge_index[1].reshape(NW, CH, CHUNK)

    xw1 = pl.pallas_call(
        _mm1_body,
        out_shape=jax.ShapeDtypeStruct((NF, 128), f32),
    )(x_flat, w1_big)

    parts1, y1, dinv = _sc_layer1(src, dst, xw1)
    parts2, y2 = _sc_layer2(src, dst, parts1, y1, dinv)

    out_flat = pl.pallas_call(
        _final_body,
        out_shape=jax.ShapeDtypeStruct((NF, 8 * D_OUT), f32),
    )(parts2, y2, dinv, w2_big)

    return out_flat.reshape(NPAD, D_OUT)[:N]


# speedup vs baseline: 1.3301x; 1.0934x over previous
import functools

import jax
import jax.numpy as jnp
from jax import lax
from jax.experimental import pallas as pl
from jax.experimental.pallas import tpu as pltpu
from jax.experimental.pallas import tpu_sc as plsc
from jax.scipy.linalg import block_diag

N = 10000
E = 320000
D_IN = 128
D_HID = 16
D_OUT = 40

NC = 2
NS = 16
NW = NC * NS

NPAD = 10240
RPT = NPAD // NS
FPT = RPT // 8
NF = NPAD // 8
XF = N // 8
CHUNK = 125
K = 8
CH = 80
CHA = 2 * CH

_mesh = plsc.VectorSubcoreMesh(
    core_axis_name="c", subcore_axis_name="s", num_cores=NC, num_subcores=NS)
_sc_params = pltpu.CompilerParams(use_tc_tiling_on_sc=False,
                                  needs_layout_passes=False)


def _rsqrt16(x):
    i = plsc.bitcast(x, jnp.int32)
    i = jnp.int32(0x5F3759DF) - lax.shift_right_logical(i, 1)
    y = plsc.bitcast(i, jnp.float32)
    for _ in range(3):
        y = y * (1.5 - 0.5 * x * y * y)
    return y


def _scatter_pass(src_v, dst_v, rows_v, ytab, atab, gsems):
    for b in range(K):
        pltpu.async_copy(ytab.at[src_v.at[b]], rows_v.at[b], gsems.at[b])

    def outer(jb, _):
        for b in range(K):
            j = jb * K + b
            pltpu.make_async_copy(
                ytab.at[src_v.at[j]], rows_v.at[b], gsems.at[b]).wait()
            pltpu.sync_copy(rows_v.at[b], atab.at[dst_v.at[j]], add=True)
            nj = j + K

            @pl.when(nj < CH)
            def _():
                pltpu.async_copy(
                    ytab.at[src_v.at[nj]], rows_v.at[b], gsems.at[b])
        return 0

    lax.fori_loop(0, CH // K, outer, 0)


def _repack_to_flat(n16_v, flat_v):
    def body(r, _):
        for g in range(8):
            flat_v[r, pl.ds(g * 16, 16)] = n16_v[r * 8 + g, :]
        return 0

    lax.fori_loop(0, FPT, body, 0)


@functools.partial(
    pl.kernel,
    out_type=(jax.ShapeDtypeStruct((NC, NF, 128), jnp.float32),
              jax.ShapeDtypeStruct((NF, 128), jnp.float32),
              jax.ShapeDtypeStruct((NF, 128), jnp.float32)),
    mesh=_mesh,
    scratch_types=[
        pltpu.VMEM((CH, CHUNK), jnp.int32),
        pltpu.VMEM((CH, CHUNK), jnp.int32),
        pltpu.VMEM((CHA, CHUNK), jnp.int32),
        pltpu.VMEM((128,), jnp.float32),
        pltpu.VMEM((RPT,), jnp.float32),
        pltpu.VMEM((RPT,), jnp.float32),
        pltpu.VMEM((16, RPT), jnp.float32),
        pltpu.VMEM((FPT, 128), jnp.float32),
        pltpu.VMEM((FPT, 128), jnp.float32),
        pltpu.VMEM((RPT, D_HID), jnp.float32),
        pltpu.VMEM((K, CHUNK, D_HID), jnp.float32),
        pltpu.VMEM_SHARED((NPAD,), jnp.float32),
        pltpu.VMEM_SHARED((NPAD, D_HID), jnp.float32),
        pltpu.VMEM_SHARED((NPAD, D_HID), jnp.float32),
        pltpu.SemaphoreType.DMA((K,)),
        pltpu.SemaphoreType.DMA((K,)),
    ],
    compiler_params=_sc_params,
)
def _sc_layer1(ei_hbm, xw_hbm,
               parts_hbm, y1_hbm, dinv_hbm,
               src_v, dst_v, dsta_v, ones_v, deg_v, dinv_v, xwt_v, xwf_v,
               dbf_v, y16_v, rows_v, degtab, ytab, atab, gsems, hsems):
    cid = lax.axis_index("c")
    sid = lax.axis_index("s")
    w = sid * NC + cid
    rs = sid * RPT
    fs = sid * FPT
    src_hbm = ei_hbm.at[0]
    dst_hbm = ei_hbm.at[1]

    def fill_ones(i, _):
        ones_v[pl.ds(i * 16, 16)] = jnp.ones((16,), jnp.float32)
        return 0

    lax.fori_loop(0, 8, fill_ones, 0)
    ones = ones_v.at[pl.ds(0, CHUNK)]

    def fill_zeros(i, _):
        deg_v[pl.ds(i * 16, 16)] = jnp.zeros((16,), jnp.float32)
        return 0

    lax.fori_loop(0, RPT // 16, fill_zeros, 0)

    pltpu.sync_copy(deg_v, degtab.at[pl.ds(rs, RPT)])
    pltpu.sync_copy(dst_hbm.at[2 * sid], dsta_v.at[pl.ds(0, CH)])
    pltpu.sync_copy(dst_hbm.at[2 * sid + 1], dsta_v.at[pl.ds(CH, CH)])
    pltpu.sync_copy(src_hbm.at[w], src_v)
    pltpu.sync_copy(dst_hbm.at[w], dst_v)
    pltpu.sync_copy(xw_hbm.at[:, pl.ds(rs, RPT)], xwt_v)
    plsc.subcore_barrier()

    for b in range(K):
        pltpu.async_copy(ones, degtab.at[dsta_v.at[b]], hsems.at[b],
                         add=True)

    def hist(jb, _):
        for b in range(K):
            j = jb * K + b
            pltpu.make_async_copy(
                ones, degtab.at[dsta_v.at[j]], hsems.at[b]).wait()
            nj = j + K

            @pl.when(nj < CHA)
            def _():
                pltpu.async_copy(ones, degtab.at[dsta_v.at[nj]],
                                 hsems.at[b], add=True)
        return 0

    lax.fori_loop(0, CHA // K, hist, 0)
    plsc.subcore_barrier()

    pltpu.sync_copy(degtab.at[pl.ds(rs, RPT)], deg_v)

    def mk_dinv(i, _):
        d = deg_v[pl.ds(i * 16, 16)]
        dinv_v[pl.ds(i * 16, 16)] = _rsqrt16(d + 1.0)
        return 0

    lax.fori_loop(0, RPT // 16, mk_dinv, 0)

    lanes = lax.iota(jnp.int32, 16)

    def scale_row(r, _):
        for g in range(8):
            n = r * 8 + g
            nn = jnp.full((16,), n, jnp.int32)
            dv = plsc.load_gather(dinv_v, [nn])
            xw = plsc.load_gather(xwt_v, [lanes, nn])
            y = dv * xw
            y16_v[n, :] = y
            xwf_v[r, pl.ds(g * 16, 16)] = y
            dbf_v[r, pl.ds(g * 16, 16)] = dv
        return 0

    lax.fori_loop(0, FPT, scale_row, 0)

    pltpu.sync_copy(y16_v, ytab.at[pl.ds(rs, RPT)])
    pltpu.sync_copy(y16_v, atab.at[pl.ds(rs, RPT)])

    @pl.when(cid == 0)
    def _():
        pltpu.sync_copy(xwf_v, y1_hbm.at[pl.ds(fs, FPT)])
        pltpu.sync_copy(dbf_v, dinv_hbm.at[pl.ds(fs, FPT)])

    plsc.subcore_barrier()
    _scatter_pass(src_v, dst_v, rows_v, ytab, atab, gsems)
    plsc.subcore_barrier()
    pltpu.sync_copy(atab.at[pl.ds(rs, RPT)], y16_v)
    _repack_to_flat(y16_v, xwf_v)
    pltpu.sync_copy(xwf_v, parts_hbm.at[cid, pl.ds(fs, FPT)])


@functools.partial(
    pl.kernel,
    out_type=(jax.ShapeDtypeStruct((NC, NF, 128), jnp.float32),
              jax.ShapeDtypeStruct((NF, 128), jnp.float32)),
    mesh=_mesh,
    scratch_types=[
        pltpu.VMEM((CH, CHUNK), jnp.int32),
        pltpu.VMEM((CH, CHUNK), jnp.int32),
        pltpu.VMEM((FPT, 128), jnp.float32),
        pltpu.VMEM((FPT, 128), jnp.float32),
        pltpu.VMEM((FPT, 128), jnp.float32),
        pltpu.VMEM((FPT, 128), jnp.float32),
        pltpu.VMEM((RPT, D_HID), jnp.float32),
        pltpu.VMEM((K, CHUNK, D_HID), jnp.float32),
        pltpu.VMEM_SHARED((NPAD, D_HID), jnp.float32),
        pltpu.VMEM_SHARED((NPAD, D_HID), jnp.float32),
        pltpu.SemaphoreType.DMA((K,)),
    ],
    compiler_params=_sc_params,
)
def _sc_layer2(ei_hbm, parts_hbm, y1_hbm, dinv_hbm,
               out_hbm, y2_hbm,
               src_v, dst_v, p0_v, p1_v, y1_v, db_v, y16_v, rows_v,
               ytab, atab, gsems):
    cid = lax.axis_index("c")
    sid = lax.axis_index("s")
    w = sid * NC + cid
    rs = sid * RPT
    fs = sid * FPT
    src_hbm = ei_hbm.at[0]
    dst_hbm = ei_hbm.at[1]

    pltpu.sync_copy(src_hbm.at[w], src_v)
    pltpu.sync_copy(dst_hbm.at[w], dst_v)
    pltpu.sync_copy(parts_hbm.at[0, pl.ds(fs, FPT)], p0_v)
    pltpu.sync_copy(parts_hbm.at[1, pl.ds(fs, FPT)], p1_v)
    pltpu.sync_copy(y1_hbm.at[pl.ds(fs, FPT)], y1_v)
    pltpu.sync_copy(dinv_hbm.at[pl.ds(fs, FPT)], db_v)

    def mk_y2(r, _):
        for g in range(8):
            s = pl.ds(g * 16, 16)
            d = db_v[r, s]
            y2 = (d * d) * (p0_v[r, s] + p1_v[r, s] - y1_v[r, s])
            y16_v[r * 8 + g, :] = y2
            p0_v[r, s] = y2
        return 0

    lax.fori_loop(0, FPT, mk_y2, 0)

    pltpu.sync_copy(y16_v, ytab.at[pl.ds(rs, RPT)])
    pltpu.sync_copy(y16_v, atab.at[pl.ds(rs, RPT)])

    @pl.when(cid == 0)
    def _():
        pltpu.sync_copy(p0_v, y2_hbm.at[pl.ds(fs, FPT)])

    plsc.subcore_barrier()
    _scatter_pass(src_v, dst_v, rows_v, ytab, atab, gsems)
    plsc.subcore_barrier()
    pltpu.sync_copy(atab.at[pl.ds(rs, RPT)], y16_v)
    _repack_to_flat(y16_v, p0_v)
    pltpu.sync_copy(p0_v, out_hbm.at[cid, pl.ds(fs, FPT)])


def _mm1_body(x_ref, w_ref, o_ref):
    r = lax.dot_general(w_ref[...], x_ref[...], (((0,), (1,)), ((), ())),
                        preferred_element_type=jnp.float32)
    o_ref[...] = jnp.concatenate(
        [r, jnp.zeros((D_HID, NPAD - N), jnp.float32)], axis=1)


def _final_body(parts_ref, y2_ref, dinv_ref, w2_ref, o_ref):
    g = dinv_ref[...] * (parts_ref[0] + parts_ref[1] - y2_ref[...])
    o_ref[...] = jnp.dot(g, w2_ref[...], preferred_element_type=jnp.float32)


def kernel(x, edge_index, W1, W2):
    f32 = jnp.float32
    w2_big = block_diag(*([W2] * 8))

    xw1t = pl.pallas_call(
        _mm1_body,
        out_shape=jax.ShapeDtypeStruct((D_HID, NPAD), f32),
    )(x, W1)

    ei4 = edge_index.reshape(2, NW, CH, CHUNK)
    parts1, y1, dinv = _sc_layer1(ei4, xw1t)
    parts2, y2 = _sc_layer2(ei4, parts1, y1, dinv)

    out_flat = pl.pallas_call(
        _final_body,
        out_shape=jax.ShapeDtypeStruct((NF, 8 * D_OUT), f32),
    )(parts2, y2, dinv, w2_big)

    return out_flat.reshape(NPAD, D_OUT)[:N]


# speedup vs baseline: 1.3495x; 1.0146x over previous
import functools

import jax
import jax.numpy as jnp
from jax import lax
from jax.experimental import pallas as pl
from jax.experimental.pallas import tpu as pltpu
from jax.experimental.pallas import tpu_sc as plsc
from jax.scipy.linalg import block_diag

N = 10000
E = 320000
D_IN = 128
D_HID = 16
D_OUT = 40

NC = 2
NS = 16
NW = NC * NS

NPAD = 10240
RPT = NPAD // NS
FPT = RPT // 8
NF = NPAD // 8
XF = N // 8
CHUNK = 125
K = 8
CH = 80
CHA = 2 * CH

_mesh = plsc.VectorSubcoreMesh(
    core_axis_name="c", subcore_axis_name="s", num_cores=NC, num_subcores=NS)
_sc_params = pltpu.CompilerParams(use_tc_tiling_on_sc=False,
                                  needs_layout_passes=False)


def _rsqrt16(x):
    i = plsc.bitcast(x, jnp.int32)
    i = jnp.int32(0x5F3759DF) - lax.shift_right_logical(i, 1)
    y = plsc.bitcast(i, jnp.float32)
    for _ in range(3):
        y = y * (1.5 - 0.5 * x * y * y)
    return y


SLOTS = 2 * K


def _scatter_pass(src_v, dst_v, rows_v, ytab, atab, gsems, ssems):
    del ssems
    for b in range(SLOTS):
        pltpu.async_copy(ytab.at[src_v.at[b]], rows_v.at[b], gsems.at[b])

    def outer(jb, _):
        for b in range(SLOTS):
            j = jb * SLOTS + b
            pltpu.make_async_copy(
                ytab.at[src_v.at[j]], rows_v.at[b], gsems.at[b]).wait()
            pltpu.sync_copy(rows_v.at[b], atab.at[dst_v.at[j]], add=True)
            nj = j + SLOTS

            @pl.when(nj < CH)
            def _():
                pltpu.async_copy(
                    ytab.at[src_v.at[nj]], rows_v.at[b], gsems.at[b])
        return 0

    lax.fori_loop(0, CH // SLOTS, outer, 0)


def _repack_to_flat(n16_v, flat_v):
    def body(r, _):
        for g in range(8):
            flat_v[r, pl.ds(g * 16, 16)] = n16_v[r * 8 + g, :]
        return 0

    lax.fori_loop(0, FPT, body, 0)


@functools.partial(
    pl.kernel,
    out_type=(jax.ShapeDtypeStruct((NC, NF, 128), jnp.float32),
              jax.ShapeDtypeStruct((NF, 128), jnp.float32),
              jax.ShapeDtypeStruct((NF, 128), jnp.float32)),
    mesh=_mesh,
    scratch_types=[
        pltpu.VMEM((CH, CHUNK), jnp.int32),
        pltpu.VMEM((CH, CHUNK), jnp.int32),
        pltpu.VMEM((CH, CHUNK), jnp.int32),
        pltpu.VMEM((128,), jnp.float32),
        pltpu.VMEM((RPT,), jnp.float32),
        pltpu.VMEM((RPT,), jnp.float32),
        pltpu.VMEM((16, RPT), jnp.float32),
        pltpu.VMEM((FPT, 128), jnp.float32),
        pltpu.VMEM((FPT, 128), jnp.float32),
        pltpu.VMEM((RPT, D_HID), jnp.float32),
        pltpu.VMEM((SLOTS, CHUNK, D_HID), jnp.float32),
        pltpu.VMEM_SHARED((NPAD,), jnp.float32),
        pltpu.VMEM_SHARED((NPAD, D_HID), jnp.float32),
        pltpu.VMEM_SHARED((NPAD, D_HID), jnp.float32),
        pltpu.SemaphoreType.DMA((SLOTS,)),
        pltpu.SemaphoreType.DMA((SLOTS,)),
        pltpu.SemaphoreType.DMA((K,)),
    ],
    compiler_params=_sc_params,
)
def _sc_layer1(ei_hbm, xw_hbm,
               parts_hbm, y1_hbm, dinv_hbm,
               src_v, dst_v, dsta_v, ones_v, deg_v, dinv_v, xwt_v, xwf_v,
               dbf_v, y16_v, rows_v, degtab, ytab, atab, gsems, ssems,
               hsems):
    cid = lax.axis_index("c")
    sid = lax.axis_index("s")
    w = sid * NC + cid
    rs = sid * RPT
    fs = sid * FPT
    src_hbm = ei_hbm.at[0]
    dst_hbm = ei_hbm.at[1]

    def fill_ones(i, _):
        ones_v[pl.ds(i * 16, 16)] = jnp.ones((16,), jnp.float32)
        return 0

    lax.fori_loop(0, 8, fill_ones, 0)
    ones = ones_v.at[pl.ds(0, CHUNK)]

    def fill_zeros(i, _):
        deg_v[pl.ds(i * 16, 16)] = jnp.zeros((16,), jnp.float32)
        return 0

    lax.fori_loop(0, RPT // 16, fill_zeros, 0)

    pltpu.sync_copy(deg_v, degtab.at[pl.ds(rs, RPT)])
    pltpu.sync_copy(src_hbm.at[w], src_v)
    pltpu.sync_copy(dst_hbm.at[w], dst_v)
    pltpu.sync_copy(xw_hbm.at[:, pl.ds(rs, RPT)], xwt_v)
    plsc.subcore_barrier()

    for p in range(2):
        pltpu.sync_copy(dst_hbm.at[2 * sid + p], dsta_v)
        for b in range(K):
            pltpu.async_copy(ones, degtab.at[dsta_v.at[b]], hsems.at[b],
                             add=True)

        def hist(jb, _):
            for b in range(K):
                j = jb * K + b
                pltpu.make_async_copy(
                    ones, degtab.at[dsta_v.at[j]], hsems.at[b]).wait()
                nj = j + K

                @pl.when(nj < CH)
                def _():
                    pltpu.async_copy(ones, degtab.at[dsta_v.at[nj]],
                                     hsems.at[b], add=True)
            return 0

        lax.fori_loop(0, CH // K, hist, 0)
    plsc.subcore_barrier()

    pltpu.sync_copy(degtab.at[pl.ds(rs, RPT)], deg_v)

    def mk_dinv(i, _):
        d = deg_v[pl.ds(i * 16, 16)]
        dinv_v[pl.ds(i * 16, 16)] = _rsqrt16(d + 1.0)
        return 0

    lax.fori_loop(0, RPT // 16, mk_dinv, 0)

    lanes = lax.iota(jnp.int32, 16)

    def scale_row(r, _):
        for g in range(8):
            n = r * 8 + g
            nn = jnp.full((16,), n, jnp.int32)
            dv = plsc.load_gather(dinv_v, [nn])
            xw = plsc.load_gather(xwt_v, [lanes, nn])
            y = dv * xw
            y16_v[n, :] = y
            xwf_v[r, pl.ds(g * 16, 16)] = y
            dbf_v[r, pl.ds(g * 16, 16)] = dv
        return 0

    lax.fori_loop(0, FPT, scale_row, 0)

    pltpu.sync_copy(y16_v, ytab.at[pl.ds(rs, RPT)])
    pltpu.sync_copy(y16_v, atab.at[pl.ds(rs, RPT)])

    @pl.when(cid == 0)
    def _():
        pltpu.sync_copy(xwf_v, y1_hbm.at[pl.ds(fs, FPT)])
        pltpu.sync_copy(dbf_v, dinv_hbm.at[pl.ds(fs, FPT)])

    plsc.subcore_barrier()
    _scatter_pass(src_v, dst_v, rows_v, ytab, atab, gsems, ssems)
    plsc.subcore_barrier()
    pltpu.sync_copy(atab.at[pl.ds(rs, RPT)], y16_v)
    _repack_to_flat(y16_v, xwf_v)
    pltpu.sync_copy(xwf_v, parts_hbm.at[cid, pl.ds(fs, FPT)])


@functools.partial(
    pl.kernel,
    out_type=(jax.ShapeDtypeStruct((NC, NF, 128), jnp.float32),
              jax.ShapeDtypeStruct((NF, 128), jnp.float32)),
    mesh=_mesh,
    scratch_types=[
        pltpu.VMEM((CH, CHUNK), jnp.int32),
        pltpu.VMEM((CH, CHUNK), jnp.int32),
        pltpu.VMEM((FPT, 128), jnp.float32),
        pltpu.VMEM((FPT, 128), jnp.float32),
        pltpu.VMEM((FPT, 128), jnp.float32),
        pltpu.VMEM((FPT, 128), jnp.float32),
        pltpu.VMEM((RPT, D_HID), jnp.float32),
        pltpu.VMEM((SLOTS, CHUNK, D_HID), jnp.float32),
        pltpu.VMEM_SHARED((NPAD, D_HID), jnp.float32),
        pltpu.VMEM_SHARED((NPAD, D_HID), jnp.float32),
        pltpu.SemaphoreType.DMA((SLOTS,)),
        pltpu.SemaphoreType.DMA((SLOTS,)),
    ],
    compiler_params=_sc_params,
)
def _sc_layer2(ei_hbm, parts_hbm, y1_hbm, dinv_hbm,
               out_hbm, y2_hbm,
               src_v, dst_v, p0_v, p1_v, y1_v, db_v, y16_v, rows_v,
               ytab, atab, gsems, ssems):
    cid = lax.axis_index("c")
    sid = lax.axis_index("s")
    w = sid * NC + cid
    rs = sid * RPT
    fs = sid * FPT
    src_hbm = ei_hbm.at[0]
    dst_hbm = ei_hbm.at[1]

    pltpu.sync_copy(src_hbm.at[w], src_v)
    pltpu.sync_copy(dst_hbm.at[w], dst_v)
    pltpu.sync_copy(parts_hbm.at[0, pl.ds(fs, FPT)], p0_v)
    pltpu.sync_copy(parts_hbm.at[1, pl.ds(fs, FPT)], p1_v)
    pltpu.sync_copy(y1_hbm.at[pl.ds(fs, FPT)], y1_v)
    pltpu.sync_copy(dinv_hbm.at[pl.ds(fs, FPT)], db_v)

    def mk_y2(r, _):
        for g in range(8):
            s = pl.ds(g * 16, 16)
            d = db_v[r, s]
            y2 = (d * d) * (p0_v[r, s] + p1_v[r, s] - y1_v[r, s])
            y16_v[r * 8 + g, :] = y2
            p0_v[r, s] = y2
        return 0

    lax.fori_loop(0, FPT, mk_y2, 0)

    pltpu.sync_copy(y16_v, ytab.at[pl.ds(rs, RPT)])
    pltpu.sync_copy(y16_v, atab.at[pl.ds(rs, RPT)])

    @pl.when(cid == 0)
    def _():
        pltpu.sync_copy(p0_v, y2_hbm.at[pl.ds(fs, FPT)])

    plsc.subcore_barrier()
    _scatter_pass(src_v, dst_v, rows_v, ytab, atab, gsems, ssems)
    plsc.subcore_barrier()
    pltpu.sync_copy(atab.at[pl.ds(rs, RPT)], y16_v)
    _repack_to_flat(y16_v, p0_v)
    pltpu.sync_copy(p0_v, out_hbm.at[cid, pl.ds(fs, FPT)])


def _mm1_body(x_ref, w_ref, o_ref):
    r = lax.dot_general(w_ref[...], x_ref[...], (((0,), (1,)), ((), ())),
                        preferred_element_type=jnp.float32)
    o_ref[...] = jnp.concatenate(
        [r, jnp.zeros((D_HID, NPAD - N), jnp.float32)], axis=1)


def _final_body(parts_ref, y2_ref, dinv_ref, w2_ref, o_ref):
    p0 = parts_ref[0, :XF]
    p1 = parts_ref[1, :XF]
    g = dinv_ref[:XF] * (p0 + p1 - y2_ref[:XF])
    o_ref[...] = jnp.dot(g, w2_ref[...], preferred_element_type=jnp.float32)


def kernel(x, edge_index, W1, W2):
    f32 = jnp.float32
    w2_big = block_diag(*([W2] * 8))

    xw1t = pl.pallas_call(
        _mm1_body,
        out_shape=jax.ShapeDtypeStruct((D_HID, NPAD), f32),
    )(x, W1)

    ei4 = edge_index.reshape(2, NW, CH, CHUNK)
    parts1, y1, dinv = _sc_layer1(ei4, xw1t)
    parts2, y2 = _sc_layer2(ei4, parts1, y1, dinv)

    out_flat = pl.pallas_call(
        _final_body,
        out_shape=jax.ShapeDtypeStruct((XF, 8 * D_OUT), f32),
    )(parts2, y2, dinv, w2_big)

    return out_flat.reshape(N, D_OUT)


# speedup vs baseline: 1.3513x; 1.0013x over previous
import functools

import jax
import jax.numpy as jnp
from jax import lax
from jax.experimental import pallas as pl
from jax.experimental.pallas import tpu as pltpu
from jax.experimental.pallas import tpu_sc as plsc
from jax.scipy.linalg import block_diag

N = 10000
E = 320000
D_IN = 128
D_HID = 16
D_OUT = 40

NC = 2
NS = 16
NW = NC * NS

NPAD = 10240
RPT = NPAD // NS
FPT = RPT // 8
NF = NPAD // 8
XF = N // 8
CHUNK = 125
K = 8
CH = 80

_mesh = plsc.VectorSubcoreMesh(
    core_axis_name="c", subcore_axis_name="s", num_cores=NC, num_subcores=NS)
_sc_params = pltpu.CompilerParams(use_tc_tiling_on_sc=False,
                                  needs_layout_passes=False)


def _rsqrt16(x):
    i = plsc.bitcast(x, jnp.int32)
    i = jnp.int32(0x5F3759DF) - lax.shift_right_logical(i, 1)
    y = plsc.bitcast(i, jnp.float32)
    for _ in range(3):
        y = y * (1.5 - 0.5 * x * y * y)
    return y


SLOTS = 2 * K


def _scatter_pass(src_v, dst_v, rows_v, ytab, atab, gsems, ssems):
    del ssems
    for b in range(SLOTS):
        pltpu.async_copy(ytab.at[src_v.at[b]], rows_v.at[b], gsems.at[b])

    def outer(jb, _):
        for b in range(SLOTS):
            j = jb * SLOTS + b
            pltpu.make_async_copy(
                ytab.at[src_v.at[j]], rows_v.at[b], gsems.at[b]).wait()
            pltpu.sync_copy(rows_v.at[b], atab.at[dst_v.at[j]], add=True)
            nj = j + SLOTS

            @pl.when(nj < CH)
            def _():
                pltpu.async_copy(
                    ytab.at[src_v.at[nj]], rows_v.at[b], gsems.at[b])
        return 0

    lax.fori_loop(0, CH // SLOTS, outer, 0)


def _repack_to_flat(n16_v, flat_v):
    def body(r, _):
        for g in range(8):
            flat_v[r, pl.ds(g * 16, 16)] = n16_v[r * 8 + g, :]
        return 0

    lax.fori_loop(0, FPT, body, 0)


@functools.partial(
    pl.kernel,
    out_type=(jax.ShapeDtypeStruct((NC, NF, 128), jnp.float32),
              jax.ShapeDtypeStruct((NF, 128), jnp.float32),
              jax.ShapeDtypeStruct((NF, 128), jnp.float32)),
    mesh=_mesh,
    scratch_types=[
        pltpu.VMEM((CH, CHUNK), jnp.int32),
        pltpu.VMEM((CH, CHUNK), jnp.int32),
        pltpu.VMEM((CH, CHUNK), jnp.int32),
        pltpu.VMEM((128,), jnp.float32),
        pltpu.VMEM((RPT,), jnp.float32),
        pltpu.VMEM((RPT,), jnp.float32),
        pltpu.VMEM((16, RPT), jnp.float32),
        pltpu.VMEM((FPT, 128), jnp.float32),
        pltpu.VMEM((FPT, 128), jnp.float32),
        pltpu.VMEM((RPT, D_HID), jnp.float32),
        pltpu.VMEM((SLOTS, CHUNK, D_HID), jnp.float32),
        pltpu.VMEM_SHARED((NPAD,), jnp.float32),
        pltpu.VMEM_SHARED((NPAD, D_HID), jnp.float32),
        pltpu.VMEM_SHARED((NPAD, D_HID), jnp.float32),
        pltpu.SemaphoreType.DMA((SLOTS,)),
        pltpu.SemaphoreType.DMA((SLOTS,)),
        pltpu.SemaphoreType.DMA((K,)),
    ],
    compiler_params=_sc_params,
)
def _sc_layer1(ei_hbm, xw_hbm,
               parts_hbm, y1_hbm, dinv_hbm,
               src_v, dst_v, dsta_v, ones_v, deg_v, dinv_v, xwt_v, xwf_v,
               dbf_v, y16_v, rows_v, degtab, ytab, atab, gsems, ssems,
               hsems):
    cid = lax.axis_index("c")
    sid = lax.axis_index("s")
    w = sid * NC + cid
    rs = sid * RPT
    fs = sid * FPT
    src_hbm = ei_hbm.at[0]
    dst_hbm = ei_hbm.at[1]

    def fill_ones(i, _):
        ones_v[pl.ds(i * 16, 16)] = jnp.ones((16,), jnp.float32)
        return 0

    lax.fori_loop(0, 8, fill_ones, 0)
    ones = ones_v.at[pl.ds(0, CHUNK)]

    def fill_zeros(i, _):
        deg_v[pl.ds(i * 16, 16)] = jnp.zeros((16,), jnp.float32)
        return 0

    lax.fori_loop(0, RPT // 16, fill_zeros, 0)

    pltpu.sync_copy(deg_v, degtab.at[pl.ds(rs, RPT)])
    pltpu.sync_copy(src_hbm.at[w], src_v)
    pltpu.sync_copy(dst_hbm.at[w], dst_v)
    pltpu.sync_copy(xw_hbm.at[:, pl.ds(rs, RPT)], xwt_v)
    plsc.subcore_barrier()

    for p in range(2):
        pltpu.sync_copy(dst_hbm.at[2 * sid + p], dsta_v)
        for b in range(K):
            pltpu.async_copy(ones, degtab.at[dsta_v.at[b]], hsems.at[b],
                             add=True)

        def hist(jb, _):
            for b in range(K):
                j = jb * K + b
                pltpu.make_async_copy(
                    ones, degtab.at[dsta_v.at[j]], hsems.at[b]).wait()
                nj = j + K

                @pl.when(nj < CH)
                def _():
                    pltpu.async_copy(ones, degtab.at[dsta_v.at[nj]],
                                     hsems.at[b], add=True)
            return 0

        lax.fori_loop(0, CH // K, hist, 0)
    plsc.subcore_barrier()

    pltpu.sync_copy(degtab.at[pl.ds(rs, RPT)], deg_v)

    def mk_dinv(i, _):
        d = deg_v[pl.ds(i * 16, 16)]
        dinv_v[pl.ds(i * 16, 16)] = _rsqrt16(d + 1.0)
        return 0

    lax.fori_loop(0, RPT // 16, mk_dinv, 0)

    lanes = lax.iota(jnp.int32, 16)

    def scale_row(r, _):
        for g in range(8):
            n = r * 8 + g
            nn = jnp.full((16,), n, jnp.int32)
            dv = plsc.load_gather(dinv_v, [nn])
            xw = plsc.load_gather(xwt_v, [lanes, nn])
            y = dv * xw
            y16_v[n, :] = y
            xwf_v[r, pl.ds(g * 16, 16)] = y
            dbf_v[r, pl.ds(g * 16, 16)] = dv
        return 0

    lax.fori_loop(0, FPT, scale_row, 0)

    pltpu.sync_copy(y16_v, ytab.at[pl.ds(rs, RPT)])
    pltpu.sync_copy(y16_v, atab.at[pl.ds(rs, RPT)])

    @pl.when(cid == 0)
    def _():
        pltpu.sync_copy(xwf_v, y1_hbm.at[pl.ds(fs, FPT)])
        pltpu.sync_copy(dbf_v, dinv_hbm.at[pl.ds(fs, FPT)])

    plsc.subcore_barrier()
    _scatter_pass(src_v, dst_v, rows_v, ytab, atab, gsems, ssems)
    plsc.subcore_barrier()
    pltpu.sync_copy(atab.at[pl.ds(rs, RPT)], y16_v)
    _repack_to_flat(y16_v, xwf_v)
    pltpu.sync_copy(xwf_v, parts_hbm.at[cid, pl.ds(fs, FPT)])


@functools.partial(
    pl.kernel,
    out_type=(jax.ShapeDtypeStruct((NC, NF, 128), jnp.float32),
              jax.ShapeDtypeStruct((NF, 128), jnp.float32)),
    mesh=_mesh,
    scratch_types=[
        pltpu.VMEM((CH, CHUNK), jnp.int32),
        pltpu.VMEM((CH, CHUNK), jnp.int32),
        pltpu.VMEM((FPT, 128), jnp.float32),
        pltpu.VMEM((FPT, 128), jnp.float32),
        pltpu.VMEM((FPT, 128), jnp.float32),
        pltpu.VMEM((FPT, 128), jnp.float32),
        pltpu.VMEM((RPT, D_HID), jnp.float32),
        pltpu.VMEM((SLOTS, CHUNK, D_HID), jnp.float32),
        pltpu.VMEM_SHARED((NPAD, D_HID), jnp.float32),
        pltpu.VMEM_SHARED((NPAD, D_HID), jnp.float32),
        pltpu.SemaphoreType.DMA((SLOTS,)),
        pltpu.SemaphoreType.DMA((SLOTS,)),
    ],
    compiler_params=_sc_params,
)
def _sc_layer2(ei_hbm, parts_hbm, y1_hbm, dinv_hbm,
               out_hbm, y2_hbm,
               src_v, dst_v, p0_v, p1_v, y1_v, db_v, y16_v, rows_v,
               ytab, atab, gsems, ssems):
    cid = lax.axis_index("c")
    sid = lax.axis_index("s")
    w = sid * NC + cid
    rs = sid * RPT
    fs = sid * FPT
    src_hbm = ei_hbm.at[0]
    dst_hbm = ei_hbm.at[1]

    pltpu.sync_copy(src_hbm.at[w], src_v)
    pltpu.sync_copy(dst_hbm.at[w], dst_v)
    pltpu.sync_copy(parts_hbm.at[0, pl.ds(fs, FPT)], p0_v)
    pltpu.sync_copy(parts_hbm.at[1, pl.ds(fs, FPT)], p1_v)
    pltpu.sync_copy(y1_hbm.at[pl.ds(fs, FPT)], y1_v)
    pltpu.sync_copy(dinv_hbm.at[pl.ds(fs, FPT)], db_v)

    def mk_y2(r, _):
        for g in range(8):
            s = pl.ds(g * 16, 16)
            d = db_v[r, s]
            y2 = (d * d) * (p0_v[r, s] + p1_v[r, s] - y1_v[r, s])
            y16_v[r * 8 + g, :] = y2
            p0_v[r, s] = y2
        return 0

    lax.fori_loop(0, FPT, mk_y2, 0)

    pltpu.sync_copy(y16_v, ytab.at[pl.ds(rs, RPT)])
    pltpu.sync_copy(y16_v, atab.at[pl.ds(rs, RPT)])

    @pl.when(cid == 0)
    def _():
        pltpu.sync_copy(p0_v, y2_hbm.at[pl.ds(fs, FPT)])

    plsc.subcore_barrier()
    _scatter_pass(src_v, dst_v, rows_v, ytab, atab, gsems, ssems)
    plsc.subcore_barrier()
    pltpu.sync_copy(atab.at[pl.ds(rs, RPT)], y16_v)
    _repack_to_flat(y16_v, p0_v)
    pltpu.sync_copy(p0_v, out_hbm.at[cid, pl.ds(fs, FPT)])


def _mm1_body(x_ref, w_ref, o_ref):
    r = lax.dot_general(w_ref[...], x_ref[...], (((0,), (1,)), ((), ())),
                        preferred_element_type=jnp.float32)
    o_ref[...] = jnp.concatenate(
        [r, jnp.zeros((D_HID, NPAD - N), jnp.float32)], axis=1)


def _final_body(parts_ref, y2_ref, dinv_ref, w2_ref, o_ref):
    p0 = parts_ref[0, :XF]
    p1 = parts_ref[1, :XF]
    g = dinv_ref[:XF] * (p0 + p1 - y2_ref[:XF])
    o_ref[...] = jnp.dot(g, w2_ref[...], preferred_element_type=jnp.float32)


def kernel(x, edge_index, W1, W2):
    f32 = jnp.float32
    w2_big = block_diag(*([W2] * 8))

    xw1t = pl.pallas_call(
        _mm1_body,
        out_shape=jax.ShapeDtypeStruct((D_HID, NPAD), f32),
    )(x, W1)

    ei4 = edge_index.reshape(2, NW, CH, CHUNK)
    parts1, y1, dinv = _sc_layer1(ei4, xw1t)
    parts2, y2 = _sc_layer2(ei4, parts1, y1, dinv)

    out_flat = pl.pallas_call(
        _final_body,
        out_shape=jax.ShapeDtypeStruct((XF, 8 * D_OUT), f32),
    )(parts2, y2, dinv, w2_big)

    return out_flat.reshape(N, D_OUT)
